# Initial kernel scaffold; baseline (speedup 1.0000x reference)
import jax
import jax.numpy as jnp
from jax.experimental import pallas as pl


def kernel(X, selected_indices, topk, pe, Wq, bq, Wk, bk, Wv, bv, Wo, bo, W1, b1, W2, b2, g1, beta1, g2, beta2):
    raise NotImplementedError("write your pallas kernel here")


# speedup vs baseline: 21.7627x; 21.7627x over previous
import functools

import jax
import jax.numpy as jnp
from jax import lax
from jax.experimental import pallas as pl
from jax.experimental.pallas import tpu as pltpu
from jax.experimental.pallas import tpu_sc as plsc

_B, _H, _W, _C = 2, 128, 128, 256
_D, _J = 64, 18
_N = _H * _W
_BN = _B * _N
_KVW = 2 * _D
_NWORK = 32
_ROWS_PW = _BN * _J // _NWORK
_SUB = 128
_CH = 256
_NCH = _ROWS_PW // _CH
_IPC = _CH // _SUB

_TILE_A = 512
_TILE_B = 256


def _qkv_body(x_ref, pe_ref, wq_ref, wkv_ref, bq_ref, bkv_ref,
              xp_ref, q_ref, kv_ref):
    xp = x_ref[...] + pe_ref[...]
    xp_ref[...] = xp
    q_ref[...] = jnp.dot(xp, wq_ref[...],
                         preferred_element_type=jnp.float32) + bq_ref[...]
    kv_ref[...] = jnp.dot(xp, wkv_ref[...],
                          preferred_element_type=jnp.float32) + bkv_ref[...]


def _layernorm(x, g, b):
    mu = jnp.mean(x, axis=-1, keepdims=True)
    xc = x - mu
    var = jnp.mean(xc * xc, axis=-1, keepdims=True)
    return xc * jax.lax.rsqrt(var + 1e-5) * g + b


def _tail_body(xp_ref, q_ref, kvsel_ref, wo_ref, bo_ref, w1_ref, b1_ref,
               w2_ref, b2_ref, g1_ref, be1_ref, g2_ref, be2_ref,
               out_ref, aw_ref):
    kv = kvsel_ref[...]
    k = kv[:, :_D].reshape(_TILE_B, _J, _D)
    v = kv[:, _D:].reshape(_TILE_B, _J, _D)
    q = q_ref[...]
    scores = jnp.sum(q[:, None, :] * k, axis=-1) * 0.125
    m = jnp.max(scores, axis=-1, keepdims=True)
    e = jnp.exp(scores - m)
    aw = e / jnp.sum(e, axis=-1, keepdims=True)
    aw_ref[...] = aw
    ctx = jnp.sum(aw[:, :, None] * v, axis=1)
    out = jnp.dot(ctx, wo_ref[...],
                  preferred_element_type=jnp.float32) + bo_ref[...]
    x1 = _layernorm(xp_ref[...] + out, g1_ref[...], be1_ref[...])
    h = jnp.maximum(jnp.dot(x1, w1_ref[...],
                            preferred_element_type=jnp.float32) + b1_ref[...],
                    0.0)
    ffn = jnp.dot(h, w2_ref[...],
                  preferred_element_type=jnp.float32) + b2_ref[...]
    out_ref[...] = _layernorm(x1 + ffn, g2_ref[...], be2_ref[...])


def _sc_gather(kv_table, idx2d):
    mesh = plsc.VectorSubcoreMesh(core_axis_name="c", subcore_axis_name="s")

    @functools.partial(
        pl.kernel,
        out_type=jax.ShapeDtypeStruct((_BN * _J, _KVW), jnp.float32),
        mesh=mesh,
        scratch_types=[
            pltpu.VMEM((2, _IPC, _SUB), jnp.int32),
            pltpu.VMEM((2, _CH, _KVW), jnp.float32),
            pltpu.SemaphoreType.DMA,
            pltpu.SemaphoreType.DMA,
        ],
    )
    def _k(kv_hbm, idx_hbm, out_hbm, idxb, kvb, sem0, sem1):
        sems = (sem0, sem1)
        wid = lax.axis_index("s") * 2 + lax.axis_index("c")
        idx_row0 = wid * (_ROWS_PW // _SUB)
        out_row0 = wid * _ROWS_PW

        def fire(c, slot):
            pltpu.sync_copy(
                idx_hbm.at[pl.ds(idx_row0 + c * _IPC, _IPC)], idxb.at[slot])
            for s in range(_IPC):
                pltpu.async_copy(kv_hbm.at[idxb.at[slot, s]],
                                 kvb.at[slot, pl.ds(s * _SUB, _SUB)],
                                 sems[slot])

        def drain_wb(c, slot):
            pltpu.make_async_copy(kv_hbm.at[pl.ds(0, _CH)], kvb.at[slot],
                                  sems[slot]).wait()
            pltpu.sync_copy(kvb.at[slot],
                            out_hbm.at[pl.ds(out_row0 + c * _CH, _CH)])

        fire(0, 0)

        def body(i, carry):
            c0 = i * 2
            for b in range(2):
                c = c0 + b

                @pl.when(c + 1 < _NCH)
                def _():
                    fire(c + 1, (b + 1) % 2)

                drain_wb(c, b)
            return carry

        lax.fori_loop(0, _NCH // 2, body, 0)

    return _k(kv_table, idx2d)


def kernel(X, selected_indices, topk, pe, Wq, bq, Wk, bk, Wv, bv, Wo, bo,
           W1, b1, W2, b2, g1, beta1, g2, beta2):
    del topk
    Xf = X.reshape(_BN, _C)
    pef = pe.reshape(_N, _C)
    Wkv = jnp.concatenate([Wk, Wv], axis=1)
    bkv = jnp.concatenate([bk, bv]).reshape(1, _KVW)
    bq2 = bq.reshape(1, _D)

    xp, q, kvt = pl.pallas_call(
        _qkv_body,
        grid=(_BN // _TILE_A,),
        in_specs=[
            pl.BlockSpec((_TILE_A, _C), lambda i: (i, 0)),
            pl.BlockSpec((_TILE_A, _C), lambda i: (i % (_N // _TILE_A), 0)),
            pl.BlockSpec((_C, _D), lambda i: (0, 0)),
            pl.BlockSpec((_C, _KVW), lambda i: (0, 0)),
            pl.BlockSpec((1, _D), lambda i: (0, 0)),
            pl.BlockSpec((1, _KVW), lambda i: (0, 0)),
        ],
        out_specs=[
            pl.BlockSpec((_TILE_A, _C), lambda i: (i, 0)),
            pl.BlockSpec((_TILE_A, _D), lambda i: (i, 0)),
            pl.BlockSpec((_TILE_A, _KVW), lambda i: (i, 0)),
        ],
        out_shape=[
            jax.ShapeDtypeStruct((_BN, _C), jnp.float32),
            jax.ShapeDtypeStruct((_BN, _D), jnp.float32),
            jax.ShapeDtypeStruct((_BN, _KVW), jnp.float32),
        ],
    )(Xf, pef, Wq, Wkv, bq2, bkv)

    offs = (jnp.arange(_B, dtype=jnp.int32) * _N)[:, None, None]
    idx2d = (selected_indices.astype(jnp.int32) + offs).reshape(
        _BN * _J // _SUB, _SUB)
    kvsel = _sc_gather(kvt, idx2d)

    out2, aw = pl.pallas_call(
        _tail_body,
        grid=(_BN // _TILE_B,),
        in_specs=[
            pl.BlockSpec((_TILE_B, _C), lambda i: (i, 0)),
            pl.BlockSpec((_TILE_B, _D), lambda i: (i, 0)),
            pl.BlockSpec((_TILE_B * _J, _KVW), lambda i: (i, 0)),
            pl.BlockSpec((_D, _C), lambda i: (0, 0)),
            pl.BlockSpec((1, _C), lambda i: (0, 0)),
            pl.BlockSpec((_C, 4 * _C), lambda i: (0, 0)),
            pl.BlockSpec((1, 4 * _C), lambda i: (0, 0)),
            pl.BlockSpec((4 * _C, _C), lambda i: (0, 0)),
            pl.BlockSpec((1, _C), lambda i: (0, 0)),
            pl.BlockSpec((1, _C), lambda i: (0, 0)),
            pl.BlockSpec((1, _C), lambda i: (0, 0)),
            pl.BlockSpec((1, _C), lambda i: (0, 0)),
            pl.BlockSpec((1, _C), lambda i: (0, 0)),
        ],
        out_specs=[
            pl.BlockSpec((_TILE_B, _C), lambda i: (i, 0)),
            pl.BlockSpec((_TILE_B, _J), lambda i: (i, 0)),
        ],
        out_shape=[
            jax.ShapeDtypeStruct((_BN, _C), jnp.float32),
            jax.ShapeDtypeStruct((_BN, _J), jnp.float32),
        ],
    )(xp, q, kvsel, Wo, bo.reshape(1, _C), W1, b1.reshape(1, 4 * _C),
      W2, b2.reshape(1, _C), g1.reshape(1, _C), beta1.reshape(1, _C),
      g2.reshape(1, _C), beta2.reshape(1, _C))

    return out2.reshape(_B, _H, _W, _C), aw.reshape(_B, _N, _J, 1)


# speedup vs baseline: 41.2149x; 1.8938x over previous
import functools

import jax
import jax.numpy as jnp
from jax import lax
from jax.experimental import pallas as pl
from jax.experimental.pallas import tpu as pltpu
from jax.experimental.pallas import tpu_sc as plsc

_B, _H, _W, _C = 2, 128, 128, 256
_D, _J = 64, 18
_N = _H * _W
_BN = _B * _N
_KVW = 2 * _D
_NWORK = 32
_QPW = _BN // _NWORK
_QPC = 16
_RPC = _QPC * _J
_NCH = _QPW // _QPC
_ILEN = 96
_IPC = _RPC // _ILEN

_TILE_A = 512
_TILE_B = 256


def _qkv_body(x_ref, pe_ref, wq_ref, wkv_ref, bq_ref, bkv_ref,
              xp_ref, q_ref, kv_ref):
    xp = x_ref[...] + pe_ref[...]
    xp_ref[...] = xp
    q_ref[...] = jnp.dot(xp, wq_ref[...],
                         preferred_element_type=jnp.float32) + bq_ref[...]
    kv_ref[...] = jnp.dot(xp, wkv_ref[...],
                          preferred_element_type=jnp.float32) + bkv_ref[...]


def _layernorm(x, g, b):
    mu = jnp.mean(x, axis=-1, keepdims=True)
    xc = x - mu
    var = jnp.mean(xc * xc, axis=-1, keepdims=True)
    return xc * jax.lax.rsqrt(var + 1e-5) * g + b


def _tail_body(xp_ref, ctx_ref, wo_ref, bo_ref, w1_ref, b1_ref,
               w2_ref, b2_ref, g1_ref, be1_ref, g2_ref, be2_ref, out_ref):
    out = jnp.dot(ctx_ref[...], wo_ref[...],
                  preferred_element_type=jnp.float32) + bo_ref[...]
    x1 = _layernorm(xp_ref[...] + out, g1_ref[...], be1_ref[...])
    h = jnp.maximum(jnp.dot(x1, w1_ref[...],
                            preferred_element_type=jnp.float32) + b1_ref[...],
                    0.0)
    ffn = jnp.dot(h, w2_ref[...],
                  preferred_element_type=jnp.float32) + b2_ref[...]
    out_ref[...] = _layernorm(x1 + ffn, g2_ref[...], be2_ref[...])


_SHUF_DNUMS = lax.GatherDimensionNumbers(
    offset_dims=(), collapsed_slice_dims=(0,), start_index_map=(0,))


def _shuf(x, idx):
    return lax.gather(x, idx[:, None], _SHUF_DNUMS, slice_sizes=(1,),
                      mode=lax.GatherScatterMode.PROMISE_IN_BOUNDS)


def _sc_attend(kv_table, idx_flat, qv):
    mesh = plsc.VectorSubcoreMesh(core_axis_name="c", subcore_axis_name="s")

    @functools.partial(
        pl.kernel,
        out_type=(
            jax.ShapeDtypeStruct((_BN, _D), jnp.float32),
            jax.ShapeDtypeStruct((_BN, 32), jnp.float32),
        ),
        mesh=mesh,
        scratch_types=[
            pltpu.VMEM((2, 8, _ILEN), jnp.int32),
            pltpu.VMEM((2, _RPC, _KVW), jnp.float32),
            pltpu.VMEM((2, _QPC, _D), jnp.float32),
            pltpu.VMEM((2, _QPC, _D), jnp.float32),
            pltpu.VMEM((2, _QPC, 32), jnp.float32),
            pltpu.SemaphoreType.DMA,
            pltpu.SemaphoreType.DMA,
        ],
    )
    def _k(kv_hbm, idx_hbm, q_hbm, ctx_hbm, aw_hbm,
           idxb, kvb, qb, ctxb, awb, sem0, sem1):
        sems = (sem0, sem1)
        wid = lax.axis_index("s") * 2 + lax.axis_index("c")
        q0 = wid * _QPW
        iota = lax.iota(jnp.int32, 16)
        xors = [iota ^ s for s in (1, 2, 4, 8)]
        msks = [(iota & s) == 0 for s in (1, 2, 4, 8)]

        def hsum(x):
            for xi in xors:
                x = x + _shuf(x, xi)
            return x

        def hmax(x):
            for xi in xors:
                x = jnp.maximum(x, _shuf(x, xi))
            return x

        def fire(c, slot):
            r0 = pl.multiple_of((wid * _NCH + c) * 8, 8)
            pltpu.sync_copy(idx_hbm.at[pl.ds(r0, 8)], idxb.at[slot])
            pltpu.sync_copy(q_hbm.at[pl.ds(q0 + c * _QPC, _QPC)], qb.at[slot])
            for s in range(_IPC):
                pltpu.async_copy(kv_hbm.at[idxb.at[slot, s]],
                                 kvb.at[slot, pl.ds(s * _ILEN, _ILEN)],
                                 sems[slot])

        def drain(slot):
            pltpu.make_async_copy(kv_hbm.at[pl.ds(0, _RPC)], kvb.at[slot],
                                  sems[slot]).wait()

        def compute(slot):
            def qbody(qi, carry):
                qr = [qb[slot, qi, pl.ds(dd * 16, 16)] * 0.125
                      for dd in range(4)]
                base = qi * _J

                def krow(j, dd):
                    return kvb[slot, base + j, pl.ds(dd * 16, 16)]

                def vrow(j, dd):
                    return kvb[slot, base + j, pl.ds(_D + dd * 16, 16)]

                def partial(j):
                    p = qr[0] * krow(j, 0)
                    for dd in range(1, 4):
                        p = p + qr[dd] * krow(j, dd)
                    return p

                vecs = [partial(j) for j in range(16)]
                for xi, m in zip(xors, msks):
                    nxt = []
                    for a, b in zip(vecs[0::2], vecs[1::2]):
                        t = a + _shuf(a, xi)
                        u = b + _shuf(b, xi)
                        nxt.append(jnp.where(m, t, u))
                    vecs = nxt
                s = vecs[0]
                s16 = hsum(partial(16))
                s17 = hsum(partial(17))
                mx = jnp.maximum(jnp.maximum(hmax(s), s16), s17)
                e = jnp.exp(s - mx)
                e16 = jnp.exp(s16 - mx)
                e17 = jnp.exp(s17 - mx)
                r = 1.0 / (hsum(e) + e16 + e17)
                aw = e * r
                aw16 = e16 * r
                aw17 = e17 * r
                awb[slot, qi, pl.ds(0, 16)] = aw
                awb[slot, qi, pl.ds(16, 16)] = jnp.where(
                    iota == 0, aw16, jnp.where(iota == 1, aw17, 0.0))
                acc = [jnp.zeros((16,), jnp.float32) for _ in range(4)]
                for j in range(16):
                    bj = _shuf(aw, jnp.full((16,), j, jnp.int32))
                    for dd in range(4):
                        acc[dd] = acc[dd] + bj * vrow(j, dd)
                for dd in range(4):
                    acc[dd] = acc[dd] + aw16 * vrow(16, dd)
                    acc[dd] = acc[dd] + aw17 * vrow(17, dd)
                for dd in range(4):
                    ctxb[slot, qi, pl.ds(dd * 16, 16)] = acc[dd]
                return carry

            lax.fori_loop(0, _QPC, qbody, 0)

        def writeback(c, slot):
            qrow = q0 + c * _QPC
            pltpu.sync_copy(ctxb.at[slot], ctx_hbm.at[pl.ds(qrow, _QPC)])
            pltpu.sync_copy(awb.at[slot], aw_hbm.at[pl.ds(qrow, _QPC)])

        fire(0, 0)

        def body(i, carry):
            c0 = i * 2
            for b in range(2):
                c = c0 + b

                @pl.when(c + 1 < _NCH)
                def _():
                    fire(c + 1, (b + 1) % 2)

                drain(b)
                compute(b)
                writeback(c, b)
            return carry

        lax.fori_loop(0, _NCH // 2, body, 0)

    return _k(kv_table, idx_flat, qv)


def kernel(X, selected_indices, topk, pe, Wq, bq, Wk, bk, Wv, bv, Wo, bo,
           W1, b1, W2, b2, g1, beta1, g2, beta2):
    del topk
    Xf = X.reshape(_BN, _C)
    pef = pe.reshape(_N, _C)
    Wkv = jnp.concatenate([Wk, Wv], axis=1)
    bkv = jnp.concatenate([bk, bv]).reshape(1, _KVW)
    bq2 = bq.reshape(1, _D)

    xp, q, kvt = pl.pallas_call(
        _qkv_body,
        grid=(_BN // _TILE_A,),
        in_specs=[
            pl.BlockSpec((_TILE_A, _C), lambda i: (i, 0)),
            pl.BlockSpec((_TILE_A, _C), lambda i: (i % (_N // _TILE_A), 0)),
            pl.BlockSpec((_C, _D), lambda i: (0, 0)),
            pl.BlockSpec((_C, _KVW), lambda i: (0, 0)),
            pl.BlockSpec((1, _D), lambda i: (0, 0)),
            pl.BlockSpec((1, _KVW), lambda i: (0, 0)),
        ],
        out_specs=[
            pl.BlockSpec((_TILE_A, _C), lambda i: (i, 0)),
            pl.BlockSpec((_TILE_A, _D), lambda i: (i, 0)),
            pl.BlockSpec((_TILE_A, _KVW), lambda i: (i, 0)),
        ],
        out_shape=[
            jax.ShapeDtypeStruct((_BN, _C), jnp.float32),
            jax.ShapeDtypeStruct((_BN, _D), jnp.float32),
            jax.ShapeDtypeStruct((_BN, _KVW), jnp.float32),
        ],
    )(Xf, pef, Wq, Wkv, bq2, bkv)

    offs = (jnp.arange(_B, dtype=jnp.int32) * _N)[:, None, None]
    idx3 = (selected_indices.astype(jnp.int32) + offs).reshape(
        _NWORK * _NCH, _IPC, _ILEN)
    idx_flat = jnp.pad(idx3, ((0, 0), (0, 8 - _IPC), (0, 0))).reshape(
        _NWORK * _NCH * 8, _ILEN)
    ctx, aw32 = _sc_attend(kvt, idx_flat, q)

    out2 = pl.pallas_call(
        _tail_body,
        grid=(_BN // _TILE_B,),
        in_specs=[
            pl.BlockSpec((_TILE_B, _C), lambda i: (i, 0)),
            pl.BlockSpec((_TILE_B, _D), lambda i: (i, 0)),
            pl.BlockSpec((_D, _C), lambda i: (0, 0)),
            pl.BlockSpec((1, _C), lambda i: (0, 0)),
            pl.BlockSpec((_C, 4 * _C), lambda i: (0, 0)),
            pl.BlockSpec((1, 4 * _C), lambda i: (0, 0)),
            pl.BlockSpec((4 * _C, _C), lambda i: (0, 0)),
            pl.BlockSpec((1, _C), lambda i: (0, 0)),
            pl.BlockSpec((1, _C), lambda i: (0, 0)),
            pl.BlockSpec((1, _C), lambda i: (0, 0)),
            pl.BlockSpec((1, _C), lambda i: (0, 0)),
            pl.BlockSpec((1, _C), lambda i: (0, 0)),
        ],
        out_specs=pl.BlockSpec((_TILE_B, _C), lambda i: (i, 0)),
        out_shape=jax.ShapeDtypeStruct((_BN, _C), jnp.float32),
    )(xp, ctx, Wo, bo.reshape(1, _C), W1, b1.reshape(1, 4 * _C),
      W2, b2.reshape(1, _C), g1.reshape(1, _C), beta1.reshape(1, _C),
      g2.reshape(1, _C), beta2.reshape(1, _C))

    aw = aw32[:, :_J]
    return out2.reshape(_B, _H, _W, _C), aw.reshape(_B, _N, _J, 1)


# speedup vs baseline: 45.9858x; 1.1158x over previous
import functools

import jax
import jax.numpy as jnp
from jax import lax
from jax.experimental import pallas as pl
from jax.experimental.pallas import tpu as pltpu
from jax.experimental.pallas import tpu_sc as plsc

_B, _H, _W, _C = 2, 128, 128, 256
_D, _J = 64, 18
_N = _H * _W
_BN = _B * _N
_KVW = 2 * _D
_NWORK = 32
_QPW = _N // _NWORK
_QPC = 16
_RPC = _QPC * _J
_NCH = _QPW // _QPC
_ILEN = 96
_SPC = _RPC // _ILEN

_TILE_A = 512
_TILE_B = 256
_GA = _N // _TILE_A
_GB = _BN // _TILE_B
_GBH = _N // _TILE_B


def _qkv_body(x_ref, pe_ref, wq_ref, wkv_ref, bq_ref, bkv_ref,
              xp_ref, q_ref, kv_ref):
    xp = x_ref[...] + pe_ref[...]
    xp_ref[...] = xp
    q_ref[...] = jnp.dot(xp, wq_ref[...],
                         preferred_element_type=jnp.float32) + bq_ref[...]
    kv_ref[...] = jnp.dot(xp, wkv_ref[...],
                          preferred_element_type=jnp.float32) + bkv_ref[...]


def _layernorm(x, g, b):
    mu = jnp.mean(x, axis=-1, keepdims=True)
    xc = x - mu
    var = jnp.mean(xc * xc, axis=-1, keepdims=True)
    return xc * jax.lax.rsqrt(var + 1e-5) * g + b


def _tail_body(xp0_ref, xp1_ref, ctx0_ref, ctx1_ref, aw0_ref, aw1_ref,
               wo_ref, bo_ref, w1_ref, b1_ref, w2_ref, b2_ref,
               g1_ref, be1_ref, g2_ref, be2_ref, out_ref, aw_ref):
    lo = pl.program_id(0) < _GBH
    xp = jnp.where(lo, xp0_ref[...], xp1_ref[...])
    ctx = jnp.where(lo, ctx0_ref[...], ctx1_ref[...])
    aw32 = jnp.where(lo, aw0_ref[...], aw1_ref[...])
    aw_ref[...] = aw32[:, :_J]
    out = jnp.dot(ctx, wo_ref[...],
                  preferred_element_type=jnp.float32) + bo_ref[...]
    x1 = _layernorm(xp + out, g1_ref[...], be1_ref[...])
    h = jnp.maximum(jnp.dot(x1.astype(jnp.bfloat16), w1_ref[...],
                            preferred_element_type=jnp.float32) + b1_ref[...],
                    0.0)
    ffn = jnp.dot(h.astype(jnp.bfloat16), w2_ref[...],
                  preferred_element_type=jnp.float32) + b2_ref[...]
    out_ref[...] = _layernorm(x1 + ffn, g2_ref[...], be2_ref[...])


_SHUF_DNUMS = lax.GatherDimensionNumbers(
    offset_dims=(), collapsed_slice_dims=(0,), start_index_map=(0,))


def _shuf(x, idx):
    return lax.gather(x, idx[:, None], _SHUF_DNUMS, slice_sizes=(1,),
                      mode=lax.GatherScatterMode.PROMISE_IN_BOUNDS)


def _sc_attend(kv_table, idx2d, qv):
    mesh = plsc.VectorSubcoreMesh(core_axis_name="c", subcore_axis_name="s")

    @functools.partial(
        pl.kernel,
        out_type=(
            jax.ShapeDtypeStruct((_N, _D), jnp.float32),
            jax.ShapeDtypeStruct((_N, 32), jnp.float32),
        ),
        mesh=mesh,
        scratch_types=[
            pltpu.VMEM((_NCH * _SPC, _ILEN), jnp.int32),
            pltpu.VMEM((2, _RPC, _KVW), jnp.float32),
            pltpu.VMEM((2, _QPC, _D), jnp.float32),
            pltpu.VMEM((2, _QPC, _D), jnp.float32),
            pltpu.VMEM((2, _QPC, 32), jnp.float32),
            pltpu.SemaphoreType.DMA,
            pltpu.SemaphoreType.DMA,
        ],
    )
    def _k(kv_hbm, idx_hbm, q_hbm, ctx_hbm, aw_hbm,
           idxb, kvb, qb, ctxb, awb, sem0, sem1):
        sems = (sem0, sem1)
        wid = lax.axis_index("s") * 2 + lax.axis_index("c")
        q0 = wid * _QPW
        iota = lax.iota(jnp.int32, 16)
        xors = [iota ^ s for s in (1, 2, 4, 8)]
        msks = [(iota & s) == 0 for s in (1, 2, 4, 8)]

        def hsum(x):
            for xi in xors:
                x = x + _shuf(x, xi)
            return x

        def hmax(x):
            for xi in xors:
                x = jnp.maximum(x, _shuf(x, xi))
            return x

        i0 = pl.multiple_of(wid * (_NCH * _SPC), 8)
        pltpu.sync_copy(idx_hbm.at[pl.ds(i0, _NCH * _SPC)], idxb)

        def fire(c, slot):
            pltpu.sync_copy(q_hbm.at[pl.ds(q0 + c * _QPC, _QPC)], qb.at[slot])
            for s in range(_SPC):
                pltpu.async_copy(kv_hbm.at[idxb.at[c * _SPC + s]],
                                 kvb.at[slot, pl.ds(s * _ILEN, _ILEN)],
                                 sems[slot])

        def drain(slot):
            pltpu.make_async_copy(kv_hbm.at[pl.ds(0, _RPC)], kvb.at[slot],
                                  sems[slot]).wait()

        def compute(slot):
            def qbody(qi, carry):
                qr = [qb[slot, qi, pl.ds(dd * 16, 16)] * 0.125
                      for dd in range(4)]
                base = qi * _J

                def partial(j):
                    p = qr[0] * kvb[slot, base + j, pl.ds(0, 16)]
                    for dd in range(1, 4):
                        p = p + qr[dd] * kvb[slot, base + j,
                                             pl.ds(dd * 16, 16)]
                    return p

                def vrow(j, dd):
                    return kvb[slot, base + j, pl.ds(_D + dd * 16, 16)]

                vecs = [partial(j) for j in range(16)]
                for xi, m in zip(xors, msks):
                    nxt = []
                    for a, b in zip(vecs[0::2], vecs[1::2]):
                        t = a + _shuf(a, xi)
                        u = b + _shuf(b, xi)
                        nxt.append(jnp.where(m, t, u))
                    vecs = nxt
                s = vecs[0]
                s16 = hsum(partial(16))
                s17 = hsum(partial(17))
                mx = jnp.maximum(jnp.maximum(hmax(s), s16), s17)
                e = jnp.exp(s - mx)
                e16 = jnp.exp(s16 - mx)
                e17 = jnp.exp(s17 - mx)
                r = 1.0 / (hsum(e) + e16 + e17)
                aw = e * r
                aw16 = e16 * r
                aw17 = e17 * r
                awb[slot, qi, pl.ds(0, 16)] = aw
                awb[slot, qi, pl.ds(16, 16)] = jnp.where(
                    iota == 0, aw16, jnp.where(iota == 1, aw17, 0.0))
                acc = [jnp.zeros((16,), jnp.float32) for _ in range(4)]
                for j in range(16):
                    bj = _shuf(aw, jnp.full((16,), j, jnp.int32))
                    for dd in range(4):
                        acc[dd] = acc[dd] + bj * vrow(j, dd)
                for j, bj in ((16, aw16), (17, aw17)):
                    for dd in range(4):
                        acc[dd] = acc[dd] + bj * vrow(j, dd)
                for dd in range(4):
                    ctxb[slot, qi, pl.ds(dd * 16, 16)] = acc[dd]
                return carry

            lax.fori_loop(0, _QPC, qbody, 0)

        def writeback(c, slot):
            qrow = q0 + c * _QPC
            pltpu.sync_copy(ctxb.at[slot], ctx_hbm.at[pl.ds(qrow, _QPC)])
            pltpu.sync_copy(awb.at[slot], aw_hbm.at[pl.ds(qrow, _QPC)])

        fire(0, 0)

        def body(i, carry):
            c0 = i * 2
            for b in range(2):
                c = c0 + b

                @pl.when(c + 1 < _NCH)
                def _():
                    fire(c + 1, (b + 1) % 2)

                drain(b)
                compute(b)
                writeback(c, b)
            return carry

        lax.fori_loop(0, _NCH // 2, body, 0)

    return _k(kv_table, idx2d, qv)


def kernel(X, selected_indices, topk, pe, Wq, bq, Wk, bk, Wv, bv, Wo, bo,
           W1, b1, W2, b2, g1, beta1, g2, beta2):
    del topk
    Xf = X.reshape(_BN, _C)
    pef = pe.reshape(_N, _C)
    Wkv = jnp.concatenate([Wk, Wv], axis=1)
    bkv = jnp.concatenate([bk, bv]).reshape(1, _KVW)
    bq2 = bq.reshape(1, _D)

    def qkv_half(h):
        return pl.pallas_call(
            _qkv_body,
            grid=(_GA,),
            in_specs=[
                pl.BlockSpec((_TILE_A, _C), lambda i: (i + h * _GA, 0)),
                pl.BlockSpec((_TILE_A, _C), lambda i: (i, 0)),
                pl.BlockSpec((_C, _D), lambda i: (0, 0)),
                pl.BlockSpec((_C, _KVW), lambda i: (0, 0)),
                pl.BlockSpec((1, _D), lambda i: (0, 0)),
                pl.BlockSpec((1, _KVW), lambda i: (0, 0)),
            ],
            out_specs=[
                pl.BlockSpec((_TILE_A, _C), lambda i: (i, 0)),
                pl.BlockSpec((_TILE_A, _D), lambda i: (i, 0)),
                pl.BlockSpec((_TILE_A, _KVW), lambda i: (i, 0)),
            ],
            out_shape=[
                jax.ShapeDtypeStruct((_N, _C), jnp.float32),
                jax.ShapeDtypeStruct((_N, _D), jnp.float32),
                jax.ShapeDtypeStruct((_N, _KVW), jnp.float32),
            ],
        )(Xf, pef, Wq, Wkv, bq2, bkv)

    xp0, q0, kv0 = qkv_half(0)
    idx0 = selected_indices[0].astype(jnp.int32).reshape(
        _N * _J // _ILEN, _ILEN)
    ctx0, aw0 = _sc_attend(kv0, idx0, q0)

    xp1, q1, kv1 = qkv_half(1)
    idx1 = selected_indices[1].astype(jnp.int32).reshape(
        _N * _J // _ILEN, _ILEN)
    ctx1, aw1 = _sc_attend(kv1, idx1, q1)

    half0 = lambda i: (jnp.minimum(i, _GBH - 1), 0)
    half1 = lambda i: (jnp.maximum(i - _GBH, 0), 0)
    const = lambda i: (0, 0)

    out2, aw = pl.pallas_call(
        _tail_body,
        grid=(_GB,),
        in_specs=[
            pl.BlockSpec((_TILE_B, _C), half0),
            pl.BlockSpec((_TILE_B, _C), half1),
            pl.BlockSpec((_TILE_B, _D), half0),
            pl.BlockSpec((_TILE_B, _D), half1),
            pl.BlockSpec((_TILE_B, 32), half0),
            pl.BlockSpec((_TILE_B, 32), half1),
            pl.BlockSpec((_D, _C), const),
            pl.BlockSpec((1, _C), const),
            pl.BlockSpec((_C, 4 * _C), const),
            pl.BlockSpec((1, 4 * _C), const),
            pl.BlockSpec((4 * _C, _C), const),
            pl.BlockSpec((1, _C), const),
            pl.BlockSpec((1, _C), const),
            pl.BlockSpec((1, _C), const),
            pl.BlockSpec((1, _C), const),
            pl.BlockSpec((1, _C), const),
        ],
        out_specs=[
            pl.BlockSpec((_TILE_B, _C), lambda i: (i, 0)),
            pl.BlockSpec((_TILE_B, _J), lambda i: (i, 0)),
        ],
        out_shape=[
            jax.ShapeDtypeStruct((_BN, _C), jnp.float32),
            jax.ShapeDtypeStruct((_BN, _J), jnp.float32),
        ],
    )(xp0, xp1, ctx0, ctx1, aw0, aw1, Wo, bo.reshape(1, _C),
      W1.astype(jnp.bfloat16), b1.reshape(1, 4 * _C),
      W2.astype(jnp.bfloat16), b2.reshape(1, _C),
      g1.reshape(1, _C), beta1.reshape(1, _C),
      g2.reshape(1, _C), beta2.reshape(1, _C))

    return out2.reshape(_B, _H, _W, _C), aw.reshape(_B, _N, _J, 1)


# speedup vs baseline: 61.2489x; 1.3319x over previous
import functools

import jax
import jax.numpy as jnp
from jax import lax
from jax.experimental import pallas as pl
from jax.experimental.pallas import tpu as pltpu
from jax.experimental.pallas import tpu_sc as plsc

_B, _H, _W, _C = 2, 128, 128, 256
_D, _J = 64, 18
_N = _H * _W
_BN = _B * _N
_KVW = 2 * _D
_NWORK = 32
_QPW = _N // _NWORK
_QPC = 16
_RPC = _QPC * _J
_NCH = _QPW // _QPC
_ILEN = 96
_SPC = _RPC // _ILEN

_TILE_A = 512
_TILE_B = 256
_GA = _N // _TILE_A
_GB = _BN // _TILE_B
_GBH = _N // _TILE_B


def _qkv_body(x_ref, pe_ref, wq_ref, wkv_ref, bq_ref, bkv_ref,
              xp_ref, q_ref, kv_ref):
    xp = x_ref[...] + pe_ref[...]
    xp_ref[...] = xp
    q_ref[...] = jnp.dot(xp, wq_ref[...],
                         preferred_element_type=jnp.float32) + bq_ref[...]
    kv_ref[...] = jnp.dot(xp, wkv_ref[...],
                          preferred_element_type=jnp.float32) + bkv_ref[...]


def _layernorm(x, g, b):
    mu = jnp.mean(x, axis=-1, keepdims=True)
    xc = x - mu
    var = jnp.mean(xc * xc, axis=-1, keepdims=True)
    return xc * jax.lax.rsqrt(var + 1e-5) * g + b


def _tail_body(xp_ref, ctx_ref, aw32_ref,
               wo_ref, bo_ref, w1_ref, b1_ref, w2_ref, b2_ref,
               g1_ref, be1_ref, g2_ref, be2_ref, out_ref, aw_ref):
    aw_ref[...] = aw32_ref[...][:, :_J]
    out = jnp.dot(ctx_ref[...], wo_ref[...],
                  preferred_element_type=jnp.float32) + bo_ref[...]
    xp = xp_ref[...]
    x1 = _layernorm(xp + out, g1_ref[...], be1_ref[...])
    h = jnp.maximum(jnp.dot(x1.astype(jnp.bfloat16), w1_ref[...],
                            preferred_element_type=jnp.float32) + b1_ref[...],
                    0.0)
    ffn = jnp.dot(h.astype(jnp.bfloat16), w2_ref[...],
                  preferred_element_type=jnp.float32) + b2_ref[...]
    out_ref[...] = _layernorm(x1 + ffn, g2_ref[...], be2_ref[...])


_SHUF_DNUMS = lax.GatherDimensionNumbers(
    offset_dims=(), collapsed_slice_dims=(0,), start_index_map=(0,))


def _shuf(x, idx):
    return lax.gather(x, idx[:, None], _SHUF_DNUMS, slice_sizes=(1,),
                      mode=lax.GatherScatterMode.PROMISE_IN_BOUNDS)


def _sc_attend(kv_table, idx2d, qv):
    mesh = plsc.VectorSubcoreMesh(core_axis_name="c", subcore_axis_name="s")

    @functools.partial(
        pl.kernel,
        out_type=(
            jax.ShapeDtypeStruct((_N, _D), jnp.float32),
            jax.ShapeDtypeStruct((_N, 32), jnp.float32),
        ),
        mesh=mesh,
        scratch_types=[
            pltpu.VMEM((_NCH * _SPC, _ILEN), jnp.int32),
            pltpu.VMEM((2, _RPC, _KVW), jnp.float32),
            pltpu.VMEM((2, _QPC, _D), jnp.float32),
            pltpu.VMEM((2, _QPC, _D), jnp.float32),
            pltpu.VMEM((2, _QPC, 32), jnp.float32),
            pltpu.SemaphoreType.DMA,
            pltpu.SemaphoreType.DMA,
            pltpu.SemaphoreType.DMA,
        ],
    )
    def _k(kv_hbm, idx_hbm, q_hbm, ctx_hbm, aw_hbm,
           idxb, kvb, qb, ctxb, awb, sem0, sem1, wbsem):
        sems = (sem0, sem1)
        wid = lax.axis_index("s") * 2 + lax.axis_index("c")
        q0 = wid * _QPW
        iota = lax.iota(jnp.int32, 16)
        xors = [iota ^ s for s in (1, 2, 4, 8)]
        msks = [(iota & s) == 0 for s in (1, 2, 4, 8)]

        def hsum(x):
            for xi in xors:
                x = x + _shuf(x, xi)
            return x

        def hmax(x):
            for xi in xors:
                x = jnp.maximum(x, _shuf(x, xi))
            return x

        i0 = pl.multiple_of(wid * (_NCH * _SPC), 8)
        pltpu.sync_copy(idx_hbm.at[pl.ds(i0, _NCH * _SPC)], idxb)

        def fire(c, slot):
            pltpu.async_copy(q_hbm.at[pl.ds(q0 + c * _QPC, _QPC)],
                             qb.at[slot], sems[slot])
            for s in range(_SPC):
                pltpu.async_copy(kv_hbm.at[idxb.at[c * _SPC + s]],
                                 kvb.at[slot, pl.ds(s * _ILEN, _ILEN)],
                                 sems[slot])

        def drain(slot):
            pltpu.make_async_copy(kv_hbm.at[pl.ds(0, _RPC)], kvb.at[slot],
                                  sems[slot]).wait()
            pltpu.make_async_copy(ctx_hbm.at[pl.ds(0, _QPC)], qb.at[slot],
                                  sems[slot]).wait()

        def wb_drain(slot):
            pltpu.make_async_copy(ctx_hbm.at[pl.ds(0, _QPC)], ctxb.at[slot],
                                  wbsem).wait()
            pltpu.make_async_copy(aw_hbm.at[pl.ds(0, _QPC)], awb.at[slot],
                                  wbsem).wait()

        def compute(c, slot):
            def one_query(slot, qi):
                qr = [qb[slot, qi, pl.ds(dd * 16, 16)] * 0.125
                      for dd in range(4)]
                base = qi * _J

                def partial(j):
                    p = qr[0] * kvb[slot, base + j, pl.ds(0, 16)]
                    for dd in range(1, 4):
                        p = p + qr[dd] * kvb[slot, base + j,
                                             pl.ds(dd * 16, 16)]
                    return p

                def vrow(j, dd):
                    return kvb[slot, base + j, pl.ds(_D + dd * 16, 16)]

                vecs = [partial(j) for j in range(16)]
                for xi, m in zip(xors, msks):
                    nxt = []
                    for a, b in zip(vecs[0::2], vecs[1::2]):
                        s1 = jnp.where(m, a, b)
                        s2 = jnp.where(m, b, a)
                        nxt.append(s1 + _shuf(s2, xi))
                    vecs = nxt
                s = vecs[0]
                s16 = hsum(partial(16))
                s17 = hsum(partial(17))
                mx = jnp.maximum(jnp.maximum(hmax(s), s16), s17)
                e = jnp.exp(s - mx)
                e16 = jnp.exp(s16 - mx)
                e17 = jnp.exp(s17 - mx)
                r = 1.0 / (hsum(e) + e16 + e17)
                aw = e * r
                aw16 = e16 * r
                aw17 = e17 * r
                awb[slot, qi, pl.ds(0, 16)] = aw
                awb[slot, qi, pl.ds(16, 16)] = jnp.where(
                    iota == 0, aw16, jnp.where(iota == 1, aw17, 0.0))
                acc = [jnp.zeros((16,), jnp.float32) for _ in range(4)]
                for j in range(16):
                    bj = _shuf(aw, jnp.full((16,), j, jnp.int32))
                    for dd in range(4):
                        acc[dd] = acc[dd] + bj * vrow(j, dd)
                for j, bj in ((16, aw16), (17, aw17)):
                    for dd in range(4):
                        acc[dd] = acc[dd] + bj * vrow(j, dd)
                for dd in range(4):
                    ctxb[slot, qi, pl.ds(dd * 16, 16)] = acc[dd]

            @pl.when(c >= 2)
            def _():
                wb_drain(slot)

            def qbody(i, carry):
                one_query(slot, i * 2)
                one_query(slot, i * 2 + 1)
                return carry

            lax.fori_loop(0, _QPC // 2, qbody, 0)

        def writeback(c, slot):
            qrow = q0 + c * _QPC
            pltpu.async_copy(ctxb.at[slot], ctx_hbm.at[pl.ds(qrow, _QPC)],
                             wbsem)
            pltpu.async_copy(awb.at[slot], aw_hbm.at[pl.ds(qrow, _QPC)],
                             wbsem)

        fire(0, 0)

        def body(i, carry):
            c0 = i * 2
            for b in range(2):
                c = c0 + b

                @pl.when(c + 1 < _NCH)
                def _():
                    fire(c + 1, (b + 1) % 2)

                drain(b)
                compute(c, b)
                writeback(c, b)
            return carry

        lax.fori_loop(0, _NCH // 2, body, 0)
        wb_drain(0)
        wb_drain(1)

    return _k(kv_table, idx2d, qv)


def kernel(X, selected_indices, topk, pe, Wq, bq, Wk, bk, Wv, bv, Wo, bo,
           W1, b1, W2, b2, g1, beta1, g2, beta2):
    del topk
    Xf = X.reshape(_BN, _C)
    pef = pe.reshape(_N, _C)
    Wkv = jnp.concatenate([Wk, Wv], axis=1)
    bkv = jnp.concatenate([bk, bv]).reshape(1, _KVW)
    bq2 = bq.reshape(1, _D)

    def qkv_half(h):
        return pl.pallas_call(
            _qkv_body,
            grid=(_GA,),
            in_specs=[
                pl.BlockSpec((_TILE_A, _C), lambda i: (i + h * _GA, 0)),
                pl.BlockSpec((_TILE_A, _C), lambda i: (i, 0)),
                pl.BlockSpec((_C, _D), lambda i: (0, 0)),
                pl.BlockSpec((_C, _KVW), lambda i: (0, 0)),
                pl.BlockSpec((1, _D), lambda i: (0, 0)),
                pl.BlockSpec((1, _KVW), lambda i: (0, 0)),
            ],
            out_specs=[
                pl.BlockSpec((_TILE_A, _C), lambda i: (i, 0)),
                pl.BlockSpec((_TILE_A, _D), lambda i: (i, 0)),
                pl.BlockSpec((_TILE_A, _KVW), lambda i: (i, 0)),
            ],
            out_shape=[
                jax.ShapeDtypeStruct((_N, _C), jnp.float32),
                jax.ShapeDtypeStruct((_N, _D), jnp.float32),
                jax.ShapeDtypeStruct((_N, _KVW), jnp.float32),
            ],
        )(Xf, pef, Wq, Wkv, bq2, bkv)

    xp0, q0, kv0 = qkv_half(0)
    idx0 = selected_indices[0].astype(jnp.int32).reshape(
        _N * _J // _ILEN, _ILEN)
    ctx0, aw0 = _sc_attend(kv0, idx0, q0)

    xp1, q1, kv1 = qkv_half(1)
    idx1 = selected_indices[1].astype(jnp.int32).reshape(
        _N * _J // _ILEN, _ILEN)
    ctx1, aw1 = _sc_attend(kv1, idx1, q1)

    const = lambda i: (0, 0)
    weights = (Wo, bo.reshape(1, _C),
               W1.astype(jnp.bfloat16), b1.reshape(1, 4 * _C),
               W2.astype(jnp.bfloat16), b2.reshape(1, _C),
               g1.reshape(1, _C), beta1.reshape(1, _C),
               g2.reshape(1, _C), beta2.reshape(1, _C))
    w_specs = [
        pl.BlockSpec((_D, _C), const),
        pl.BlockSpec((1, _C), const),
        pl.BlockSpec((_C, 4 * _C), const),
        pl.BlockSpec((1, 4 * _C), const),
        pl.BlockSpec((4 * _C, _C), const),
        pl.BlockSpec((1, _C), const),
        pl.BlockSpec((1, _C), const),
        pl.BlockSpec((1, _C), const),
        pl.BlockSpec((1, _C), const),
        pl.BlockSpec((1, _C), const),
    ]
    out_shapes = [
        jax.ShapeDtypeStruct((_BN, _C), jnp.float32),
        jax.ShapeDtypeStruct((_BN, _J), jnp.float32),
    ]

    def tail_half(h, xp_h, ctx_h, aw_h, prev):
        def body(*refs):
            _tail_body(*refs[2:]) if prev is not None else _tail_body(*refs)

        half_specs = [
            pl.BlockSpec((_TILE_B, _C), lambda i: (i, 0)),
            pl.BlockSpec((_TILE_B, _D), lambda i: (i, 0)),
            pl.BlockSpec((_TILE_B, 32), lambda i: (i, 0)),
        ]
        out_specs = [
            pl.BlockSpec((_TILE_B, _C), lambda i: (i + h * _GBH, 0)),
            pl.BlockSpec((_TILE_B, _J), lambda i: (i + h * _GBH, 0)),
        ]
        if prev is None:
            return pl.pallas_call(
                body, grid=(_GBH,),
                in_specs=half_specs + w_specs,
                out_specs=out_specs, out_shape=out_shapes,
            )(xp_h, ctx_h, aw_h, *weights)
        return pl.pallas_call(
            body, grid=(_GBH,),
            in_specs=[pl.BlockSpec(memory_space=pl.ANY),
                      pl.BlockSpec(memory_space=pl.ANY)] + half_specs + w_specs,
            out_specs=out_specs, out_shape=out_shapes,
            input_output_aliases={0: 0, 1: 1},
        )(prev[0], prev[1], xp_h, ctx_h, aw_h, *weights)

    out2_0, aw_0 = tail_half(0, xp0, ctx0, aw0, None)
    out2, aw = tail_half(1, xp1, ctx1, aw1, (out2_0, aw_0))

    return out2.reshape(_B, _H, _W, _C), aw.reshape(_B, _N, _J, 1)
